# baseline (speedup 1.0000x reference)
import functools

import jax
import jax.numpy as jnp
from jax import lax
from jax.experimental import pallas as pl
from jax.experimental.pallas import tpu as pltpu
from jax.experimental.pallas import tpu_sc as plsc

N_NODES = 10000
N_EDGES = 320000
FEAT = 128

NUM_CORES = 2
NUM_SUBCORES = 16
HFEAT = FEAT // NUM_CORES
EDGES_PER_TILE = N_EDGES // NUM_SUBCORES
CHUNK = 400
N_CHUNKS = EDGES_PER_TILE // CHUNK
N_PAIRS = N_CHUNKS // 2
ROWS_PER_TILE = 624
TAIL_ROW0 = ROWS_PER_TILE * NUM_SUBCORES
TAIL_ROWS = N_NODES - TAIL_ROW0
DEG_W = 16


@functools.partial(
    pl.kernel,
    out_type=(
        jax.ShapeDtypeStruct((NUM_CORES, N_NODES, HFEAT), jnp.float32),
        jax.ShapeDtypeStruct((NUM_CORES, N_NODES, DEG_W), jnp.float32),
    ),
    mesh=plsc.VectorSubcoreMesh(core_axis_name="c", subcore_axis_name="s"),
    compiler_params=pltpu.CompilerParams(use_tc_tiling_on_sc=False),
    scratch_types=(
        pltpu.VMEM((4, CHUNK), jnp.int32),
        pltpu.VMEM((4, CHUNK), jnp.int32),
        pltpu.VMEM((CHUNK, HFEAT), jnp.float32),
        pltpu.VMEM((CHUNK, HFEAT), jnp.float32),
        pltpu.VMEM((CHUNK, DEG_W), jnp.float32),
        pltpu.VMEM_SHARED((N_NODES, HFEAT), jnp.float32),
        pltpu.VMEM_SHARED((N_NODES, DEG_W), jnp.float32),
        pltpu.SemaphoreType.DMA,
        pltpu.SemaphoreType.DMA,
        pltpu.SemaphoreType.DMA,
        pltpu.SemaphoreType.DMA,
        pltpu.SemaphoreType.DMA,
        pltpu.SemaphoreType.DMA,
    ),
)
def _edge_aggregate(x_hbm, edata_hbm, zeros_hbm, zdeg_hbm,
                    a_out, deg_out,
                    idx0, idx1, rows0, rows1, ones_v,
                    accum_sh, deg_sh, g0, g1, s0, s1, d0, d1):
    cid = lax.axis_index("c")
    sid = lax.axis_index("s")

    row0 = sid * ROWS_PER_TILE
    pltpu.sync_copy(zeros_hbm.at[pl.ds(row0, ROWS_PER_TILE)],
                    accum_sh.at[pl.ds(row0, ROWS_PER_TILE)])

    pltpu.sync_copy(zdeg_hbm.at[pl.ds(row0, ROWS_PER_TILE)],
                    deg_sh.at[pl.ds(row0, ROWS_PER_TILE)])

    @pl.when(sid == NUM_SUBCORES - 1)
    def _zero_tail():
        pltpu.sync_copy(zeros_hbm.at[pl.ds(TAIL_ROW0, TAIL_ROWS)],
                        accum_sh.at[pl.ds(TAIL_ROW0, TAIL_ROWS)])

        pltpu.sync_copy(zdeg_hbm.at[pl.ds(TAIL_ROW0, TAIL_ROWS)],
                        deg_sh.at[pl.ds(TAIL_ROW0, TAIL_ROWS)])

    onehot = jnp.where(lax.iota(jnp.int32, 16) == 0,
                       jnp.float32(1.0), jnp.float32(0.0))

    def fill(r, carry):
        ones_v[r, :] = onehot
        return carry
    lax.fori_loop(0, CHUNK, fill, 0)

    plsc.subcore_barrier()

    def load_idx(c, idx_b):
        pltpu.sync_copy(edata_hbm.at[sid, c], idx_b)

    def issue_gather(idx_b, rows_b, sem):
        pltpu.async_copy(x_hbm.at[idx_b.at[cid]], rows_b, sem)

    def wait_gather(idx_b, rows_b, sem):
        pltpu.make_async_copy(x_hbm.at[idx_b.at[cid]], rows_b, sem).wait()

    def issue_scatter(idx_b, rows_b, sem):
        pltpu.async_copy(rows_b, accum_sh.at[idx_b.at[2]], sem, add=True)

    def wait_scatter(idx_b, rows_b, sem):
        pltpu.make_async_copy(rows_b, accum_sh.at[idx_b.at[2]], sem).wait()

    def count_here(c):
        return jnp.where(cid == 0, c < N_CHUNKS // 2, c >= N_CHUNKS // 2)

    def issue_deg(c, idx_b, sem):
        @pl.when(count_here(c))
        def _d():
            pltpu.async_copy(ones_v, deg_sh.at[idx_b.at[2]], sem, add=True)

    def wait_deg(c, idx_b, sem):
        @pl.when(count_here(c))
        def _w():
            pltpu.make_async_copy(ones_v, deg_sh.at[idx_b.at[2]], sem).wait()

    def scale(idx_b, rows_b):
        @plsc.parallel_loop(0, CHUNK // 16, unroll=2)
        def grp(g):
            ew16 = idx_b[3, pl.ds(g * 16, 16)]
            for e in range(16):
                w16 = jnp.full(
                    (16,), lax.bitcast_convert_type(ew16[e], jnp.float32),
                    jnp.float32)
                r = g * 16 + e
                for j in range(HFEAT // 16):
                    rows_b[r, pl.ds(j * 16, 16)] = (
                        rows_b[r, pl.ds(j * 16, 16)] * w16)

    load_idx(0, idx0)
    issue_gather(idx0, rows0, g0)

    def pair(t, carry):
        cA = 2 * t
        wait_gather(idx0, rows0, g0)

        @pl.when(t >= 1)
        def _free1():
            wait_scatter(idx1, rows1, s1)
            wait_deg(cA - 1, idx1, d1)

        load_idx(cA + 1, idx1)
        issue_gather(idx1, rows1, g1)
        scale(idx0, rows0)
        issue_scatter(idx0, rows0, s0)
        issue_deg(cA, idx0, d0)

        wait_gather(idx1, rows1, g1)
        wait_scatter(idx0, rows0, s0)
        wait_deg(cA, idx0, d0)

        @pl.when(t < N_PAIRS - 1)
        def _next0():
            load_idx(cA + 2, idx0)
            issue_gather(idx0, rows0, g0)

        scale(idx1, rows1)
        issue_scatter(idx1, rows1, s1)
        issue_deg(cA + 1, idx1, d1)
        return carry

    lax.fori_loop(0, N_PAIRS, pair, 0)

    wait_scatter(idx1, rows1, s1)
    wait_deg(N_CHUNKS - 1, idx1, d1)

    plsc.subcore_barrier()

    pltpu.sync_copy(accum_sh.at[pl.ds(row0, ROWS_PER_TILE)],
                    a_out.at[cid, pl.ds(row0, ROWS_PER_TILE)])

    pltpu.sync_copy(deg_sh.at[pl.ds(row0, ROWS_PER_TILE)],
                    deg_out.at[cid, pl.ds(row0, ROWS_PER_TILE)])

    @pl.when(sid == NUM_SUBCORES - 1)
    def _write_tail():
        pltpu.sync_copy(accum_sh.at[pl.ds(TAIL_ROW0, TAIL_ROWS)],
                        a_out.at[cid, pl.ds(TAIL_ROW0, TAIL_ROWS)])

        pltpu.sync_copy(deg_sh.at[pl.ds(TAIL_ROW0, TAIL_ROWS)],
                        deg_out.at[cid, pl.ds(TAIL_ROW0, TAIL_ROWS)])


BLK = 2000


def _dense_body(a_ref, d_ref, x_ref, wn_ref, wl_ref, we_ref, norm_ref,
                out_ref):
    a = jnp.concatenate([a_ref[0], a_ref[1]], axis=-1)
    h = jnp.dot(a, wn_ref[...], preferred_element_type=jnp.float32)
    h = h * norm_ref[...]
    deg = d_ref[0] + d_ref[1]
    has_in = deg[:, 0:1] > 0.0
    ls = jnp.dot(x_ref[...], wl_ref[...], preferred_element_type=jnp.float32)
    le = jnp.dot(x_ref[...], we_ref[...], preferred_element_type=jnp.float32)
    z = h + jnp.where(has_in, ls, le)
    out_ref[...] = jnp.where(z >= 0.0, z, z * jnp.float32(0.01))


def kernel(x, edge_index, edge_weight, norm, weight_neighbor, loop_weight,
           evolve_loop_weight):
    src = edge_index[0].astype(jnp.int32)
    dst = edge_index[1].astype(jnp.int32)
    ew_bits = lax.bitcast_convert_type(edge_weight, jnp.int32)
    src2 = src * 2
    edata = (jnp.stack([src2, src2 + 1, dst, ew_bits])
             .reshape(4, NUM_SUBCORES, N_CHUNKS, CHUNK)
             .transpose(1, 2, 0, 3))
    x_r = x.reshape(2 * N_NODES, HFEAT)
    zeros = jnp.zeros((N_NODES, HFEAT), jnp.float32)
    zdeg = jnp.zeros((N_NODES, DEG_W), jnp.float32)

    a_parts, deg = _edge_aggregate(x_r, edata, zeros, zdeg)

    a_spec = pl.BlockSpec((NUM_CORES, BLK, HFEAT), lambda i: (0, i, 0))
    row_spec = pl.BlockSpec((BLK, FEAT), lambda i: (i, 0))
    deg_spec = pl.BlockSpec((NUM_CORES, BLK, DEG_W), lambda i: (0, i, 0))
    w_spec = pl.BlockSpec((FEAT, FEAT), lambda i: (0, 0))
    norm_spec = pl.BlockSpec((BLK, 1), lambda i: (i, 0))

    out = pl.pallas_call(
        _dense_body,
        grid=(N_NODES // BLK,),
        in_specs=[a_spec, deg_spec, row_spec,
                  w_spec, w_spec, w_spec, norm_spec],
        out_specs=row_spec,
        out_shape=jax.ShapeDtypeStruct((N_NODES, FEAT), jnp.float32),
    )(a_parts, deg, x, weight_neighbor, loop_weight, evolve_loop_weight,
      norm)
    return out


# speedup vs baseline: 1.0109x; 1.0109x over previous
import functools

import jax
import jax.numpy as jnp
from jax import lax
from jax.experimental import pallas as pl
from jax.experimental.pallas import tpu as pltpu
from jax.experimental.pallas import tpu_sc as plsc

N_NODES = 10000
N_EDGES = 320000
FEAT = 128

NUM_CORES = 2
NUM_SUBCORES = 16
HFEAT = FEAT // NUM_CORES
EDGES_PER_TILE = N_EDGES // NUM_SUBCORES
CHUNK = 400
N_CHUNKS = EDGES_PER_TILE // CHUNK
N_PAIRS = N_CHUNKS // 2
ROWS_PER_TILE = 624
TAIL_ROW0 = ROWS_PER_TILE * NUM_SUBCORES
TAIL_ROWS = N_NODES - TAIL_ROW0
DEG_W = 16


@functools.partial(
    pl.kernel,
    out_type=(
        jax.ShapeDtypeStruct((NUM_CORES, N_NODES, HFEAT), jnp.float32),
        jax.ShapeDtypeStruct((N_NODES, DEG_W), jnp.float32),
    ),
    mesh=plsc.VectorSubcoreMesh(core_axis_name="c", subcore_axis_name="s"),
    compiler_params=pltpu.CompilerParams(use_tc_tiling_on_sc=False),
    scratch_types=(
        pltpu.VMEM((4, CHUNK), jnp.int32),
        pltpu.VMEM((4, CHUNK), jnp.int32),
        pltpu.VMEM((CHUNK, HFEAT), jnp.float32),
        pltpu.VMEM((CHUNK, HFEAT), jnp.float32),
        pltpu.VMEM((CHUNK, DEG_W), jnp.float32),
        pltpu.VMEM_SHARED((N_NODES, HFEAT), jnp.float32),
        pltpu.VMEM_SHARED((N_NODES, DEG_W), jnp.float32),
        pltpu.SemaphoreType.DMA,
        pltpu.SemaphoreType.DMA,
        pltpu.SemaphoreType.DMA,
        pltpu.SemaphoreType.DMA,
        pltpu.SemaphoreType.DMA,
        pltpu.SemaphoreType.DMA,
    ),
)
def _edge_aggregate(x_hbm, edata_hbm, zeros_hbm, zdeg_hbm,
                    a_out, deg_out,
                    idx0, idx1, rows0, rows1, ones_v,
                    accum_sh, deg_sh, g0, g1, s0, s1, d0, d1):
    cid = lax.axis_index("c")
    sid = lax.axis_index("s")

    row0 = sid * ROWS_PER_TILE
    pltpu.sync_copy(zeros_hbm.at[pl.ds(row0, ROWS_PER_TILE)],
                    accum_sh.at[pl.ds(row0, ROWS_PER_TILE)])

    @pl.when(cid == 0)
    def _zero_deg():
        pltpu.sync_copy(zdeg_hbm.at[pl.ds(row0, ROWS_PER_TILE)],
                        deg_sh.at[pl.ds(row0, ROWS_PER_TILE)])

    @pl.when(sid == NUM_SUBCORES - 1)
    def _zero_tail():
        pltpu.sync_copy(zeros_hbm.at[pl.ds(TAIL_ROW0, TAIL_ROWS)],
                        accum_sh.at[pl.ds(TAIL_ROW0, TAIL_ROWS)])

        @pl.when(cid == 0)
        def _zero_deg_tail():
            pltpu.sync_copy(zdeg_hbm.at[pl.ds(TAIL_ROW0, TAIL_ROWS)],
                            deg_sh.at[pl.ds(TAIL_ROW0, TAIL_ROWS)])

    onehot = jnp.where(lax.iota(jnp.int32, 16) == 0,
                       jnp.float32(1.0), jnp.float32(0.0))

    def fill(r, carry):
        ones_v[r, :] = onehot
        return carry
    lax.fori_loop(0, CHUNK, fill, 0)

    plsc.subcore_barrier()

    def load_idx(c, idx_b):
        pltpu.sync_copy(edata_hbm.at[sid, c], idx_b)

    def issue_gather(idx_b, rows_b, sem):
        pltpu.async_copy(x_hbm.at[idx_b.at[cid]], rows_b, sem)

    def wait_gather(idx_b, rows_b, sem):
        pltpu.make_async_copy(x_hbm.at[idx_b.at[cid]], rows_b, sem).wait()

    def issue_scatter(idx_b, rows_b, sem):
        pltpu.async_copy(rows_b, accum_sh.at[idx_b.at[2]], sem, add=True)

    def wait_scatter(idx_b, rows_b, sem):
        pltpu.make_async_copy(rows_b, accum_sh.at[idx_b.at[2]], sem).wait()

    def issue_deg(idx_b, sem):
        @pl.when(cid == 0)
        def _d():
            pltpu.async_copy(ones_v, deg_sh.at[idx_b.at[2]], sem, add=True)

    def wait_deg(idx_b, sem):
        @pl.when(cid == 0)
        def _w():
            pltpu.make_async_copy(ones_v, deg_sh.at[idx_b.at[2]], sem).wait()

    def scale(idx_b, rows_b):
        @plsc.parallel_loop(0, CHUNK // 16, unroll=2)
        def grp(g):
            ew16 = idx_b[3, pl.ds(g * 16, 16)]
            for e in range(16):
                w16 = jnp.full(
                    (16,), lax.bitcast_convert_type(ew16[e], jnp.float32),
                    jnp.float32)
                r = g * 16 + e
                for j in range(HFEAT // 16):
                    rows_b[r, pl.ds(j * 16, 16)] = (
                        rows_b[r, pl.ds(j * 16, 16)] * w16)

    load_idx(0, idx0)
    issue_gather(idx0, rows0, g0)

    def pair(t, carry):
        cA = 2 * t
        wait_gather(idx0, rows0, g0)

        @pl.when(t >= 1)
        def _free1():
            wait_scatter(idx1, rows1, s1)
            wait_deg(idx1, d1)

        load_idx(cA + 1, idx1)
        issue_gather(idx1, rows1, g1)
        scale(idx0, rows0)
        issue_scatter(idx0, rows0, s0)
        issue_deg(idx0, d0)

        wait_gather(idx1, rows1, g1)
        wait_scatter(idx0, rows0, s0)
        wait_deg(idx0, d0)

        @pl.when(t < N_PAIRS - 1)
        def _next0():
            load_idx(cA + 2, idx0)
            issue_gather(idx0, rows0, g0)

        scale(idx1, rows1)
        issue_scatter(idx1, rows1, s1)
        issue_deg(idx1, d1)
        return carry

    lax.fori_loop(0, N_PAIRS, pair, 0)

    wait_scatter(idx1, rows1, s1)
    wait_deg(idx1, d1)

    plsc.subcore_barrier()

    pltpu.sync_copy(accum_sh.at[pl.ds(row0, ROWS_PER_TILE)],
                    a_out.at[cid, pl.ds(row0, ROWS_PER_TILE)])

    @pl.when(cid == 0)
    def _write_deg():
        pltpu.sync_copy(deg_sh.at[pl.ds(row0, ROWS_PER_TILE)],
                        deg_out.at[pl.ds(row0, ROWS_PER_TILE)])

    @pl.when(sid == NUM_SUBCORES - 1)
    def _write_tail():
        pltpu.sync_copy(accum_sh.at[pl.ds(TAIL_ROW0, TAIL_ROWS)],
                        a_out.at[cid, pl.ds(TAIL_ROW0, TAIL_ROWS)])

        @pl.when(cid == 0)
        def _write_deg_tail():
            pltpu.sync_copy(deg_sh.at[pl.ds(TAIL_ROW0, TAIL_ROWS)],
                            deg_out.at[pl.ds(TAIL_ROW0, TAIL_ROWS)])


BLK = 2000


def _dense_body(a_ref, d_ref, x_ref, wn_ref, wl_ref, we_ref, norm_ref,
                out_ref):
    a = jnp.concatenate([a_ref[0], a_ref[1]], axis=-1)
    h = jnp.dot(a, wn_ref[...], preferred_element_type=jnp.float32)
    h = h * norm_ref[...]
    has_in = d_ref[...][:, 0:1] > 0.0
    ls = jnp.dot(x_ref[...], wl_ref[...], preferred_element_type=jnp.float32)
    le = jnp.dot(x_ref[...], we_ref[...], preferred_element_type=jnp.float32)
    z = h + jnp.where(has_in, ls, le)
    out_ref[...] = jnp.where(z >= 0.0, z, z * jnp.float32(0.01))


def kernel(x, edge_index, edge_weight, norm, weight_neighbor, loop_weight,
           evolve_loop_weight):
    src = edge_index[0].astype(jnp.int32)
    dst = edge_index[1].astype(jnp.int32)
    ew_bits = lax.bitcast_convert_type(edge_weight, jnp.int32)
    src2 = src * 2
    edata = (jnp.stack([src2, src2 + 1, dst, ew_bits])
             .reshape(4, NUM_SUBCORES, N_CHUNKS, CHUNK)
             .transpose(1, 2, 0, 3))
    x_r = x.reshape(2 * N_NODES, HFEAT)
    zeros = jnp.zeros((N_NODES, HFEAT), jnp.float32)
    zdeg = jnp.zeros((N_NODES, DEG_W), jnp.float32)

    a_parts, deg = _edge_aggregate(x_r, edata, zeros, zdeg)

    a_spec = pl.BlockSpec((NUM_CORES, BLK, HFEAT), lambda i: (0, i, 0))
    row_spec = pl.BlockSpec((BLK, FEAT), lambda i: (i, 0))
    deg_spec = pl.BlockSpec((BLK, DEG_W), lambda i: (i, 0))
    w_spec = pl.BlockSpec((FEAT, FEAT), lambda i: (0, 0))
    norm_spec = pl.BlockSpec((BLK, 1), lambda i: (i, 0))

    out = pl.pallas_call(
        _dense_body,
        grid=(N_NODES // BLK,),
        in_specs=[a_spec, deg_spec, row_spec,
                  w_spec, w_spec, w_spec, norm_spec],
        out_specs=row_spec,
        out_shape=jax.ShapeDtypeStruct((N_NODES, FEAT), jnp.float32),
    )(a_parts, deg, x, weight_neighbor, loop_weight, evolve_loop_weight,
      norm)
    return out


# speedup vs baseline: 1.0681x; 1.0566x over previous
import functools

import jax
import jax.numpy as jnp
from jax import lax
from jax.experimental import pallas as pl
from jax.experimental.pallas import tpu as pltpu
from jax.experimental.pallas import tpu_sc as plsc

N_NODES = 10000
N_EDGES = 320000
FEAT = 128

NUM_CORES = 2
NUM_SUBCORES = 16
HFEAT = FEAT // NUM_CORES
EDGES_PER_TILE = N_EDGES // NUM_SUBCORES
CHUNK = 400
N_CHUNKS = EDGES_PER_TILE // CHUNK
N_PAIRS = N_CHUNKS // 2
ROWS_PER_TILE = 624
TAIL_ROW0 = ROWS_PER_TILE * NUM_SUBCORES
TAIL_ROWS = N_NODES - TAIL_ROW0
DEG_W = 16


@functools.partial(
    pl.kernel,
    out_type=(
        jax.ShapeDtypeStruct((NUM_CORES, N_NODES, HFEAT), jnp.float32),
        jax.ShapeDtypeStruct((N_NODES, DEG_W), jnp.float32),
    ),
    mesh=plsc.VectorSubcoreMesh(core_axis_name="c", subcore_axis_name="s"),
    compiler_params=pltpu.CompilerParams(use_tc_tiling_on_sc=False),
    scratch_types=(
        pltpu.VMEM((3, CHUNK), jnp.int32),
        pltpu.VMEM((3, CHUNK), jnp.int32),
        pltpu.VMEM((CHUNK, HFEAT), jnp.float32),
        pltpu.VMEM((CHUNK, HFEAT), jnp.float32),
        pltpu.VMEM((CHUNK, DEG_W), jnp.float32),
        pltpu.VMEM_SHARED((N_NODES, HFEAT), jnp.float32),
        pltpu.VMEM_SHARED((N_NODES, DEG_W), jnp.float32),
        pltpu.SemaphoreType.DMA,
        pltpu.SemaphoreType.DMA,
        pltpu.SemaphoreType.DMA,
        pltpu.SemaphoreType.DMA,
        pltpu.SemaphoreType.DMA,
        pltpu.SemaphoreType.DMA,
    ),
)
def _edge_aggregate(xl_hbm, xr_hbm, edata_hbm, zeros_hbm, zdeg_hbm,
                    a_out, deg_out,
                    idx0, idx1, rows0, rows1, ones_v,
                    accum_sh, deg_sh, g0, g1, s0, s1, d0, d1):
    cid = lax.axis_index("c")
    sid = lax.axis_index("s")

    row0 = sid * ROWS_PER_TILE
    pltpu.sync_copy(zeros_hbm.at[pl.ds(row0, ROWS_PER_TILE)],
                    accum_sh.at[pl.ds(row0, ROWS_PER_TILE)])

    @pl.when(cid == 0)
    def _zero_deg():
        pltpu.sync_copy(zdeg_hbm.at[pl.ds(row0, ROWS_PER_TILE)],
                        deg_sh.at[pl.ds(row0, ROWS_PER_TILE)])

    @pl.when(sid == NUM_SUBCORES - 1)
    def _zero_tail():
        pltpu.sync_copy(zeros_hbm.at[pl.ds(TAIL_ROW0, TAIL_ROWS)],
                        accum_sh.at[pl.ds(TAIL_ROW0, TAIL_ROWS)])

        @pl.when(cid == 0)
        def _zero_deg_tail():
            pltpu.sync_copy(zdeg_hbm.at[pl.ds(TAIL_ROW0, TAIL_ROWS)],
                            deg_sh.at[pl.ds(TAIL_ROW0, TAIL_ROWS)])

    onehot = jnp.where(lax.iota(jnp.int32, 16) == 0,
                       jnp.float32(1.0), jnp.float32(0.0))

    def fill(r, carry):
        ones_v[r, :] = onehot
        return carry
    lax.fori_loop(0, CHUNK, fill, 0)

    plsc.subcore_barrier()

    def load_idx(c, idx_b):
        pltpu.sync_copy(edata_hbm.at[sid, c], idx_b)

    def issue_gather(idx_b, rows_b, sem):
        @pl.when(cid == 0)
        def _gl():
            pltpu.async_copy(xl_hbm.at[idx_b.at[0]], rows_b, sem)

        @pl.when(cid == 1)
        def _gr():
            pltpu.async_copy(xr_hbm.at[idx_b.at[0]], rows_b, sem)

    def wait_gather(idx_b, rows_b, sem):
        pltpu.make_async_copy(xl_hbm.at[idx_b.at[0]], rows_b, sem).wait()

    def issue_scatter(idx_b, rows_b, sem):
        pltpu.async_copy(rows_b, accum_sh.at[idx_b.at[1]], sem, add=True)

    def wait_scatter(idx_b, rows_b, sem):
        pltpu.make_async_copy(rows_b, accum_sh.at[idx_b.at[1]], sem).wait()

    def issue_deg(idx_b, sem):
        @pl.when(cid == 0)
        def _d():
            pltpu.async_copy(ones_v, deg_sh.at[idx_b.at[1]], sem, add=True)

    def wait_deg(idx_b, sem):
        @pl.when(cid == 0)
        def _w():
            pltpu.make_async_copy(ones_v, deg_sh.at[idx_b.at[1]], sem).wait()

    def scale(idx_b, rows_b):
        @plsc.parallel_loop(0, CHUNK // 16, unroll=2)
        def grp(g):
            ew16 = idx_b[2, pl.ds(g * 16, 16)]
            for e in range(16):
                w16 = jnp.full(
                    (16,), lax.bitcast_convert_type(ew16[e], jnp.float32),
                    jnp.float32)
                r = g * 16 + e
                for j in range(HFEAT // 16):
                    rows_b[r, pl.ds(j * 16, 16)] = (
                        rows_b[r, pl.ds(j * 16, 16)] * w16)

    load_idx(0, idx0)
    issue_gather(idx0, rows0, g0)

    def pair(t, carry):
        cA = 2 * t
        wait_gather(idx0, rows0, g0)

        @pl.when(t >= 1)
        def _free1():
            wait_scatter(idx1, rows1, s1)
            wait_deg(idx1, d1)

        load_idx(cA + 1, idx1)
        issue_gather(idx1, rows1, g1)
        scale(idx0, rows0)
        issue_scatter(idx0, rows0, s0)
        issue_deg(idx0, d0)

        wait_gather(idx1, rows1, g1)
        wait_scatter(idx0, rows0, s0)
        wait_deg(idx0, d0)

        @pl.when(t < N_PAIRS - 1)
        def _next0():
            load_idx(cA + 2, idx0)
            issue_gather(idx0, rows0, g0)

        scale(idx1, rows1)
        issue_scatter(idx1, rows1, s1)
        issue_deg(idx1, d1)
        return carry

    lax.fori_loop(0, N_PAIRS, pair, 0)

    wait_scatter(idx1, rows1, s1)
    wait_deg(idx1, d1)

    plsc.subcore_barrier()

    pltpu.sync_copy(accum_sh.at[pl.ds(row0, ROWS_PER_TILE)],
                    a_out.at[cid, pl.ds(row0, ROWS_PER_TILE)])

    @pl.when(cid == 0)
    def _write_deg():
        pltpu.sync_copy(deg_sh.at[pl.ds(row0, ROWS_PER_TILE)],
                        deg_out.at[pl.ds(row0, ROWS_PER_TILE)])

    @pl.when(sid == NUM_SUBCORES - 1)
    def _write_tail():
        pltpu.sync_copy(accum_sh.at[pl.ds(TAIL_ROW0, TAIL_ROWS)],
                        a_out.at[cid, pl.ds(TAIL_ROW0, TAIL_ROWS)])

        @pl.when(cid == 0)
        def _write_deg_tail():
            pltpu.sync_copy(deg_sh.at[pl.ds(TAIL_ROW0, TAIL_ROWS)],
                            deg_out.at[pl.ds(TAIL_ROW0, TAIL_ROWS)])


BLK = 2000


def _dense_body(a_ref, d_ref, x_ref, wn_ref, wl_ref, we_ref, norm_ref,
                out_ref):
    a = jnp.concatenate([a_ref[0], a_ref[1]], axis=-1)
    h = jnp.dot(a, wn_ref[...], preferred_element_type=jnp.float32)
    h = h * norm_ref[...]
    has_in = d_ref[...][:, 0:1] > 0.0
    ls = jnp.dot(x_ref[...], wl_ref[...], preferred_element_type=jnp.float32)
    le = jnp.dot(x_ref[...], we_ref[...], preferred_element_type=jnp.float32)
    z = h + jnp.where(has_in, ls, le)
    out_ref[...] = jnp.where(z >= 0.0, z, z * jnp.float32(0.01))


def kernel(x, edge_index, edge_weight, norm, weight_neighbor, loop_weight,
           evolve_loop_weight):
    src = edge_index[0].astype(jnp.int32)
    dst = edge_index[1].astype(jnp.int32)
    ew_bits = lax.bitcast_convert_type(edge_weight, jnp.int32)
    edata = (jnp.stack([src, dst, ew_bits])
             .reshape(3, NUM_SUBCORES, N_CHUNKS, CHUNK)
             .transpose(1, 2, 0, 3))
    xl = x[:, :HFEAT]
    xr = x[:, HFEAT:]
    zeros = jnp.zeros((N_NODES, HFEAT), jnp.float32)
    zdeg = jnp.zeros((N_NODES, DEG_W), jnp.float32)

    a_parts, deg = _edge_aggregate(xl, xr, edata, zeros, zdeg)

    a_spec = pl.BlockSpec((NUM_CORES, BLK, HFEAT), lambda i: (0, i, 0))
    row_spec = pl.BlockSpec((BLK, FEAT), lambda i: (i, 0))
    deg_spec = pl.BlockSpec((BLK, DEG_W), lambda i: (i, 0))
    w_spec = pl.BlockSpec((FEAT, FEAT), lambda i: (0, 0))
    norm_spec = pl.BlockSpec((BLK, 1), lambda i: (i, 0))

    out = pl.pallas_call(
        _dense_body,
        grid=(N_NODES // BLK,),
        in_specs=[a_spec, deg_spec, row_spec,
                  w_spec, w_spec, w_spec, norm_spec],
        out_specs=row_spec,
        out_shape=jax.ShapeDtypeStruct((N_NODES, FEAT), jnp.float32),
    )(a_parts, deg, x, weight_neighbor, loop_weight, evolve_loop_weight,
      norm)
    return out
